# baseline (speedup 1.0000x reference)
import functools

import jax
import jax.numpy as jnp
from jax.experimental import pallas as pl
from jax.experimental.pallas import tpu as pltpu


def _ca_kernel(x_ref, w1t_ref, b1_ref, w2_ref, b2_ref, o_ref, *, inv_hw):
    for i in range(2):
        x = x_ref[i]
        s2 = jnp.sum(x, axis=2, dtype=jnp.float32)
        pooled = jnp.sum(s2, axis=1, keepdims=True) * inv_hw
        h = jnp.sum(w1t_ref[...] * pooled, axis=0, keepdims=True) + b1_ref[...]
        h = jnp.maximum(h, 0.0)
        y = jnp.sum(w2_ref[...] * h, axis=1, keepdims=True) + b2_ref[...]
        scale = jax.nn.sigmoid(y)
        o_ref[i] = x * scale[:, :, None]


def kernel(x, w1, b1, w2, b2):
    B, C, H, W = x.shape
    hidden = w1.shape[0]
    f32 = jnp.float32

    out = pl.pallas_call(
        functools.partial(_ca_kernel, inv_hw=1.0 / (H * W)),
        out_shape=jax.ShapeDtypeStruct((B, C, H, W), x.dtype),
        grid=(B // 2,),
        in_specs=[
            pl.BlockSpec((2, C, H, W), lambda b: (b, 0, 0, 0)),
            pl.BlockSpec((C, hidden), lambda b: (0, 0)),
            pl.BlockSpec((1, hidden), lambda b: (0, 0)),
            pl.BlockSpec((C, hidden), lambda b: (0, 0)),
            pl.BlockSpec((C, 1), lambda b: (0, 0)),
        ],
        out_specs=pl.BlockSpec((2, C, H, W), lambda b: (b, 0, 0, 0)),
        compiler_params=pltpu.CompilerParams(
            dimension_semantics=("parallel",),
            vmem_limit_bytes=48 << 20),
    )(x, w1.T.astype(f32), b1.reshape(1, hidden).astype(f32),
      w2.astype(f32), b2.reshape(C, 1).astype(f32))
    return out


# speedup vs baseline: 1.0019x; 1.0019x over previous
import functools

import jax
import jax.numpy as jnp
from jax.experimental import pallas as pl
from jax.experimental.pallas import tpu as pltpu


def _ca_body(x, w1t, b1, w2, b2, inv_hw):
    s2 = jnp.sum(x, axis=2, dtype=jnp.float32)
    pooled = jnp.sum(s2, axis=1, keepdims=True) * inv_hw
    h = jnp.sum(w1t * pooled, axis=0, keepdims=True) + b1
    h = jnp.maximum(h, 0.0)
    y = jnp.sum(w2 * h, axis=1, keepdims=True) + b2
    scale = jax.nn.sigmoid(y)
    return x * scale[:, :, None]


def _ca_kernel(xa_ref, xb_ref, w1t_ref, b1_ref, w2_ref, b2_ref, o_ref, *,
               inv_hw):
    w1t = w1t_ref[...]
    b1 = b1_ref[...]
    w2 = w2_ref[...]
    b2 = b2_ref[...]
    o_ref[0] = _ca_body(xa_ref[0], w1t, b1, w2, b2, inv_hw)
    o_ref[1] = _ca_body(xb_ref[0], w1t, b1, w2, b2, inv_hw)


def kernel(x, w1, b1, w2, b2):
    B, C, H, W = x.shape
    hidden = w1.shape[0]
    f32 = jnp.float32

    out = pl.pallas_call(
        functools.partial(_ca_kernel, inv_hw=1.0 / (H * W)),
        out_shape=jax.ShapeDtypeStruct((B, C, H, W), x.dtype),
        grid=(B // 2,),
        in_specs=[
            pl.BlockSpec((1, C, H, W), lambda b: (2 * b, 0, 0, 0)),
            pl.BlockSpec((1, C, H, W), lambda b: (2 * b + 1, 0, 0, 0)),
            pl.BlockSpec((C, hidden), lambda b: (0, 0)),
            pl.BlockSpec((1, hidden), lambda b: (0, 0)),
            pl.BlockSpec((C, hidden), lambda b: (0, 0)),
            pl.BlockSpec((C, 1), lambda b: (0, 0)),
        ],
        out_specs=pl.BlockSpec((2, C, H, W), lambda b: (b, 0, 0, 0)),
        compiler_params=pltpu.CompilerParams(
            dimension_semantics=("parallel",),
            vmem_limit_bytes=48 << 20),
    )(x, x, w1.T.astype(f32), b1.reshape(1, hidden).astype(f32),
      w2.astype(f32), b2.reshape(C, 1).astype(f32))
    return out
